# baseline (speedup 1.0000x reference)
import functools

import jax
import jax.numpy as jnp
from jax import lax
from jax.experimental import pallas as pl
from jax.experimental.pallas import tpu as pltpu
from jax.experimental.pallas import tpu_sc as plsc

N = 10000
E = 320000
H = 128
L = 3

NC = 2
NS = 16
NW = NC * NS
EW = E // NW
CH = 80
NCH = EW // CH
NSL = 4
NP = 10240
NPT = NP // NS
ZCH = 64

H2 = H // 2

BRN = 2000
BRE = 2000


def _ln(t, g, be):
    mu = jnp.mean(t, axis=-1, keepdims=True)
    var = jnp.mean((t - mu) * (t - mu), axis=-1, keepdims=True)
    return (t - mu) * lax.rsqrt(var + 1e-5) * g + be


def _dot(a, b):
    return jnp.dot(a, b, preferred_element_type=jnp.float32)


def _enc_body(x_ref, w1_ref, w2_ref, w3_ref, aux_ref, o_ref):
    b1, b2, b3 = aux_ref[0:1, :], aux_ref[1:2, :], aux_ref[2:3, :]
    g, be = aux_ref[3:4, :], aux_ref[4:5, :]
    t = jnp.maximum(_dot(x_ref[...], w1_ref[...]) + b1, 0.0)
    t = jnp.maximum(_dot(t, w2_ref[...]) + b2, 0.0)
    t = _dot(t, w3_ref[...]) + b3
    o_ref[...] = _ln(t, g, be)


def _mlp3(x, w1, w2, w3, aux, br):
    n, din = x.shape
    return pl.pallas_call(
        _enc_body,
        grid=(n // br,),
        in_specs=[
            pl.BlockSpec((br, din), lambda i: (i, 0)),
            pl.BlockSpec((din, H), lambda i: (0, 0)),
            pl.BlockSpec((H, H), lambda i: (0, 0)),
            pl.BlockSpec((H, H), lambda i: (0, 0)),
            pl.BlockSpec((8, H), lambda i: (0, 0)),
        ],
        out_specs=pl.BlockSpec((br, H), lambda i: (i, 0)),
        out_shape=jax.ShapeDtypeStruct((n, H), jnp.float32),
    )(x, w1, w2, w3, aux)


def _ab_body(h_ref, wd_ref, ws_ref, a_ref, b_ref):
    a_ref[...] = _dot(h_ref[...], wd_ref[...])
    b_ref[...] = _dot(h_ref[...], ws_ref[...])


def _ab(h, wd, ws):
    return pl.pallas_call(
        _ab_body,
        grid=(N // BRN,),
        in_specs=[
            pl.BlockSpec((BRN, H), lambda i: (i, 0)),
            pl.BlockSpec((H, H), lambda i: (0, 0)),
            pl.BlockSpec((H, H), lambda i: (0, 0)),
        ],
        out_specs=[
            pl.BlockSpec((BRN, H), lambda i: (i, 0)),
            pl.BlockSpec((BRN, H), lambda i: (i, 0)),
        ],
        out_shape=[
            jax.ShapeDtypeStruct((N, H), jnp.float32),
            jax.ShapeDtypeStruct((N, H), jnp.float32),
        ],
    )(h, wd, ws)


def _edge_body(g_ref, e_ref, w1_ref, w2_ref, w3_ref, aux_ref, o_ref):
    b1, b2, b3 = aux_ref[0:1, :], aux_ref[1:2, :], aux_ref[2:3, :]
    g, be = aux_ref[3:4, :], aux_ref[4:5, :]
    e = e_ref[...]
    t = g_ref[...] + _dot(e, w1_ref[...]) + b1
    t = jnp.maximum(t, 0.0)
    t = jnp.maximum(_dot(t, w2_ref[...]) + b2, 0.0)
    t = _dot(t, w3_ref[...]) + b3
    o_ref[...] = e + _ln(t, g, be)


def _edge_mlp(g, e, w1, w2, w3, aux):
    return pl.pallas_call(
        _edge_body,
        grid=(E // BRE,),
        in_specs=[
            pl.BlockSpec((BRE, H), lambda i: (i, 0)),
            pl.BlockSpec((BRE, H), lambda i: (i, 0)),
            pl.BlockSpec((H, H), lambda i: (0, 0)),
            pl.BlockSpec((H, H), lambda i: (0, 0)),
            pl.BlockSpec((H, H), lambda i: (0, 0)),
            pl.BlockSpec((8, H), lambda i: (0, 0)),
        ],
        out_specs=pl.BlockSpec((BRE, H), lambda i: (i, 0)),
        out_shape=jax.ShapeDtypeStruct((E, H), jnp.float32),
    )(g, e, w1, w2, w3, aux)


def _node_body(h_ref, p0_ref, p1_ref, wh_ref, wa_ref, w2_ref, w3_ref, aux_ref, o_ref):
    b1, b2, b3 = aux_ref[0:1, :], aux_ref[1:2, :], aux_ref[2:3, :]
    g, be = aux_ref[3:4, :], aux_ref[4:5, :]
    h = h_ref[...]
    agg = p0_ref[...] + p1_ref[...]
    t = _dot(h, wh_ref[...]) + _dot(agg, wa_ref[...]) + b1
    t = jnp.maximum(t, 0.0)
    t = jnp.maximum(_dot(t, w2_ref[...]) + b2, 0.0)
    t = _dot(t, w3_ref[...]) + b3
    o_ref[...] = h + _ln(t, g, be)


def _node_mlp(h, p0, p1, wh, wa, w2, w3, aux):
    return pl.pallas_call(
        _node_body,
        grid=(N // BRN,),
        in_specs=[
            pl.BlockSpec((BRN, H), lambda i: (i, 0)),
            pl.BlockSpec((BRN, H), lambda i: (i, 0)),
            pl.BlockSpec((BRN, H), lambda i: (i, 0)),
            pl.BlockSpec((H, H), lambda i: (0, 0)),
            pl.BlockSpec((H, H), lambda i: (0, 0)),
            pl.BlockSpec((H, H), lambda i: (0, 0)),
            pl.BlockSpec((H, H), lambda i: (0, 0)),
            pl.BlockSpec((8, H), lambda i: (0, 0)),
        ],
        out_specs=pl.BlockSpec((BRN, H), lambda i: (i, 0)),
        out_shape=jax.ShapeDtypeStruct((N, H), jnp.float32),
    )(h, p0, p1, wh, wa, w2, w3, aux)


def _dec_body(h_ref, w1_ref, w2_ref, w3_ref, aux_ref, o_ref):
    b1, b2, b3 = aux_ref[0:1, :], aux_ref[1:2, :], aux_ref[2:3, :]
    t = jnp.maximum(_dot(h_ref[...], w1_ref[...]) + b1, 0.0)
    t = jnp.maximum(_dot(t, w2_ref[...]) + b2, 0.0)
    o_ref[...] = _dot(t, w3_ref[...]) + b3


def _dec(h, w1, w2, w3, aux):
    return pl.pallas_call(
        _dec_body,
        grid=(N // BRN,),
        in_specs=[
            pl.BlockSpec((BRN, H), lambda i: (i, 0)),
            pl.BlockSpec((H, H), lambda i: (0, 0)),
            pl.BlockSpec((H, H), lambda i: (0, 0)),
            pl.BlockSpec((H, H), lambda i: (0, 0)),
            pl.BlockSpec((8, H), lambda i: (0, 0)),
        ],
        out_specs=pl.BlockSpec((BRN, H), lambda i: (i, 0)),
        out_shape=jax.ShapeDtypeStruct((N, H), jnp.float32),
    )(h, w1, w2, w3, aux)


def _sc_gather_add(a, b, dst, src):
    mesh = plsc.VectorSubcoreMesh(core_axis_name="c", subcore_axis_name="s")
    LA = NSL - 1

    @functools.partial(
        pl.kernel,
        mesh=mesh,
        out_type=jax.ShapeDtypeStruct((E, H), jnp.float32),
        scratch_types=(
            pltpu.VMEM((EW,), jnp.int32),
            pltpu.VMEM((EW,), jnp.int32),
            pltpu.VMEM((NSL, CH, H), jnp.float32),
            pltpu.VMEM((NSL, CH, H), jnp.float32),
            pltpu.SemaphoreType.DMA((NSL,)),
            pltpu.SemaphoreType.DMA((NSL,)),
            pltpu.SemaphoreType.DMA((NSL,)),
        ),
    )
    def k(a_hbm, b_hbm, d_hbm, s_hbm, g_hbm, idxd, idxs, abuf, bbuf, sga, sgb, sw):
        w = lax.axis_index("s") * NC + lax.axis_index("c")
        ebase = w * EW
        pltpu.sync_copy(d_hbm.at[pl.ds(ebase, EW)], idxd)
        pltpu.sync_copy(s_hbm.at[pl.ds(ebase, EW)], idxs)

        def start(j, slot):
            pltpu.async_copy(a_hbm.at[idxd.at[pl.ds(j * CH, CH)]],
                             abuf.at[slot], sga.at[slot])
            pltpu.async_copy(b_hbm.at[idxs.at[pl.ds(j * CH, CH)]],
                             bbuf.at[slot], sgb.at[slot])

        def finish(j, slot):
            pltpu.make_async_copy(a_hbm.at[idxd.at[pl.ds(0, CH)]],
                                  abuf.at[slot], sga.at[slot]).wait()
            pltpu.make_async_copy(b_hbm.at[idxs.at[pl.ds(0, CH)]],
                                  bbuf.at[slot], sgb.at[slot]).wait()

            def addrows(r, carry):
                for rr in range(8):
                    for t in range(H // 16):
                        s_ = pl.ds(t * 16, 16)
                        plsc.addupdate(abuf.at[slot, r * 8 + rr, s_],
                                       bbuf[slot, r * 8 + rr, s_])
                return carry

            lax.fori_loop(0, CH // 8, addrows, 0)
            pltpu.async_copy(abuf.at[slot], g_hbm.at[pl.ds(ebase + j * CH, CH)],
                             sw.at[slot])

        def drain_w(slot):
            pltpu.make_async_copy(abuf.at[slot], g_hbm.at[pl.ds(0, CH)],
                                  sw.at[slot]).wait()

        for j0 in range(LA):
            start(j0, j0)

        def body(j, carry):
            for sl in range(NSL):
                @pl.when(j % NSL == sl)
                def _step(sl=sl):
                    @pl.when(j >= NSL)
                    def _dw():
                        drain_w(sl)
                    start(j, sl)
                    finish(j - LA, (sl + 1) % NSL)
            return carry

        lax.fori_loop(LA, NCH, body, 0)
        for jt in range(NCH - LA, NCH):
            finish(jt, jt % NSL)
        for sl in range(NSL):
            drain_w(sl)

    return k(a, b, dst, src)


def _sc_scatter(e, src2dp):
    mesh = plsc.VectorSubcoreMesh(core_axis_name="c", subcore_axis_name="s")

    @functools.partial(
        pl.kernel,
        mesh=mesh,
        out_type=jax.ShapeDtypeStruct((NC * NP, H), jnp.float32),
        scratch_types=(
            pltpu.VMEM((NCH + 11, CH), jnp.int32),
            pltpu.VMEM((CH, H), jnp.float32),
            pltpu.VMEM((CH, H), jnp.float32),
            pltpu.VMEM((ZCH, H), jnp.float32),
            pltpu.VMEM_SHARED((NP, H), jnp.float32),
            pltpu.SemaphoreType.DMA,
            pltpu.SemaphoreType.DMA,
        ),
    )
    def k(e_hbm, s_hbm, o_hbm, idxv, rows0, rows1, obuf, acc, sl0, sl1):
        c = lax.axis_index("c")
        s = lax.axis_index("s")
        w = c * NS + s

        def zb(r, carry):
            for t in range(H // 16):
                obuf[r, pl.ds(t * 16, 16)] = jnp.zeros((16,), jnp.float32)
            return carry

        lax.fori_loop(0, ZCH, zb, 0)
        for t in range(NPT // ZCH):
            pltpu.sync_copy(obuf, acc.at[pl.ds(s * NPT + t * ZCH, ZCH)])
        plsc.subcore_barrier()

        rbase = w * NCH
        rb8 = (rbase // 8) * 8
        off = rbase - rb8
        pltpu.sync_copy(s_hbm.at[pl.ds(rb8, NCH + 11)], idxv)

        rows, sl = (rows0, rows1), (sl0, sl1)

        def start(j, slot):
            pltpu.async_copy(e_hbm.at[pl.ds(w * EW + j * CH, CH)], rows[slot], sl[slot])

        def add(j, slot):
            pltpu.make_async_copy(e_hbm.at[pl.ds(0, CH)], rows[slot], sl[slot]).wait()
            pltpu.sync_copy(rows[slot], acc.at[idxv.at[off + j]], add=True)

        start(0, 0)

        def body(j, carry):
            @pl.when(j % 2 == 1)
            def odd():
                start(j, 1)
                add(j - 1, 0)

            @pl.when(j % 2 == 0)
            def even():
                start(j, 0)
                add(j - 1, 1)

            return carry

        lax.fori_loop(1, NCH, body, 0)
        add(NCH - 1, 0)
        plsc.subcore_barrier()

        for t in range(NPT // ZCH):
            pltpu.sync_copy(acc.at[pl.ds(s * NPT + t * ZCH, ZCH)], obuf)
            pltpu.sync_copy(obuf, o_hbm.at[pl.ds(c * NP + s * NPT + t * ZCH, ZCH)])

    return k(e, src2dp)


def _aux(b1, b2, b3, g=None, be=None):
    z = jnp.zeros_like(b1)
    g = z if g is None else g
    be = z if be is None else be
    return jnp.stack([b1, b2, b3, g, be, z, z, z])


def kernel(x, edge_index, edge_attr, mean_vec_x, std_vec_x, mean_vec_edge,
           std_vec_edge, params):
    p = params
    f32 = jnp.float32

    w1n = p['ne_W1'] / std_vec_x[:, None]
    b1n = p['ne_b1'] - (mean_vec_x / std_vec_x) @ p['ne_W1']
    w1n = jnp.pad(w1n, ((0, 16 - w1n.shape[0]), (0, 0)))
    xp = jnp.pad(x.astype(f32), ((0, 0), (0, 16 - x.shape[1])))
    h = _mlp3(xp, w1n, p['ne_W2'], p['ne_W3'],
              _aux(b1n, p['ne_b2'], p['ne_b3'], p['ne_g'], p['ne_be']), BRN)

    w1e = p['ee_W1'] / std_vec_edge[:, None]
    b1e = p['ee_b1'] - (mean_vec_edge / std_vec_edge) @ p['ee_W1']
    w1e = jnp.pad(w1e, ((0, 8 - w1e.shape[0]), (0, 0)))
    eap = jnp.pad(edge_attr.astype(f32), ((0, 0), (0, 8 - edge_attr.shape[1])))
    e = _mlp3(eap, w1e, p['ee_W2'], p['ee_W3'],
              _aux(b1e, p['ee_b2'], p['ee_b3'], p['ee_g'], p['ee_be']), BRE)

    src = edge_index[0].astype(jnp.int32)
    dst = edge_index[1].astype(jnp.int32)
    src2dp = jnp.pad(src.reshape(E // CH, CH), ((0, 8), (0, 0)))

    for l in range(L):
        w1 = p['ep_W1'][l]
        a, b = _ab(h, w1[:H], w1[H:2 * H])
        g = _sc_gather_add(a, b, dst, src)
        e = _edge_mlp(g, e, w1[2 * H:], p['ep_W2'][l], p['ep_W3'][l],
                      _aux(p['ep_b1'][l], p['ep_b2'][l], p['ep_b3'][l],
                           p['ep_g'][l], p['ep_be'][l]))
        part = _sc_scatter(e, src2dp)
        nw1 = p['np_W1'][l]
        h = _node_mlp(h, part[:N], part[NP:NP + N], nw1[:H], nw1[H:],
                      p['np_W2'][l], p['np_W3'][l],
                      _aux(p['np_b1'][l], p['np_b2'][l], p['np_b3'][l],
                           p['np_g'][l], p['np_be'][l]))

    w3p = jnp.pad(p['dec_W3'], ((0, 0), (0, H - p['dec_W3'].shape[1])))
    b3p = jnp.pad(p['dec_b3'], ((0, H - p['dec_b3'].shape[0]),))
    out = _dec(h, p['dec_W1'], p['dec_W2'], w3p,
               _aux(p['dec_b1'], p['dec_b2'], b3p))
    return out[:, :2]


# speedup vs baseline: 1.0050x; 1.0050x over previous
import functools

import jax
import jax.numpy as jnp
from jax import lax
from jax.experimental import pallas as pl
from jax.experimental.pallas import tpu as pltpu
from jax.experimental.pallas import tpu_sc as plsc

N = 10000
E = 320000
H = 128
L = 3

NC = 2
NS = 16
NW = NC * NS
EW = E // NW
CH = 80
NCH = EW // CH
NSL = 4
NP = 10240
NPT = NP // NS
ZCH = 64

H2 = H // 2

BRN = 2000
BRE = 2000


def _ln(t, g, be):
    mu = jnp.mean(t, axis=-1, keepdims=True)
    var = jnp.mean((t - mu) * (t - mu), axis=-1, keepdims=True)
    return (t - mu) * lax.rsqrt(var + 1e-5) * g + be


def _dot(a, b):
    return jnp.dot(a, b, preferred_element_type=jnp.float32)


def _enc_body(x_ref, w1_ref, w2_ref, w3_ref, aux_ref, o_ref):
    b1, b2, b3 = aux_ref[0:1, :], aux_ref[1:2, :], aux_ref[2:3, :]
    g, be = aux_ref[3:4, :], aux_ref[4:5, :]
    t = jnp.maximum(_dot(x_ref[...], w1_ref[...]) + b1, 0.0)
    t = jnp.maximum(_dot(t, w2_ref[...]) + b2, 0.0)
    t = _dot(t, w3_ref[...]) + b3
    o_ref[...] = _ln(t, g, be)


def _mlp3(x, w1, w2, w3, aux, br):
    n, din = x.shape
    return pl.pallas_call(
        _enc_body,
        grid=(n // br,),
        in_specs=[
            pl.BlockSpec((br, din), lambda i: (i, 0)),
            pl.BlockSpec((din, H), lambda i: (0, 0)),
            pl.BlockSpec((H, H), lambda i: (0, 0)),
            pl.BlockSpec((H, H), lambda i: (0, 0)),
            pl.BlockSpec((8, H), lambda i: (0, 0)),
        ],
        out_specs=pl.BlockSpec((br, H), lambda i: (i, 0)),
        out_shape=jax.ShapeDtypeStruct((n, H), jnp.float32),
    )(x, w1, w2, w3, aux)


def _ab_body(h_ref, wd_ref, ws_ref, a_ref, b_ref):
    a_ref[...] = _dot(h_ref[...], wd_ref[...])
    b_ref[...] = _dot(h_ref[...], ws_ref[...])


def _ab(h, wd, ws):
    return pl.pallas_call(
        _ab_body,
        grid=(N // BRN,),
        in_specs=[
            pl.BlockSpec((BRN, H), lambda i: (i, 0)),
            pl.BlockSpec((H, H), lambda i: (0, 0)),
            pl.BlockSpec((H, H), lambda i: (0, 0)),
        ],
        out_specs=[
            pl.BlockSpec((BRN, H), lambda i: (i, 0)),
            pl.BlockSpec((BRN, H), lambda i: (i, 0)),
        ],
        out_shape=[
            jax.ShapeDtypeStruct((N, H), jnp.float32),
            jax.ShapeDtypeStruct((N, H), jnp.float32),
        ],
    )(h, wd, ws)


def _edge_body(g_ref, e_ref, w1_ref, w2_ref, w3_ref, aux_ref, o_ref):
    b1, b2, b3 = aux_ref[0:1, :], aux_ref[1:2, :], aux_ref[2:3, :]
    g, be = aux_ref[3:4, :], aux_ref[4:5, :]
    e = e_ref[...]
    t = g_ref[...] + _dot(e, w1_ref[...]) + b1
    t = jnp.maximum(t, 0.0)
    t = jnp.maximum(_dot(t, w2_ref[...]) + b2, 0.0)
    t = _dot(t, w3_ref[...]) + b3
    o_ref[...] = e + _ln(t, g, be)


def _edge_mlp(g, e, w1, w2, w3, aux):
    return pl.pallas_call(
        _edge_body,
        grid=(E // BRE,),
        in_specs=[
            pl.BlockSpec((BRE, H), lambda i: (i, 0)),
            pl.BlockSpec((BRE, H), lambda i: (i, 0)),
            pl.BlockSpec((H, H), lambda i: (0, 0)),
            pl.BlockSpec((H, H), lambda i: (0, 0)),
            pl.BlockSpec((H, H), lambda i: (0, 0)),
            pl.BlockSpec((8, H), lambda i: (0, 0)),
        ],
        out_specs=pl.BlockSpec((BRE, H), lambda i: (i, 0)),
        out_shape=jax.ShapeDtypeStruct((E, H), jnp.float32),
    )(g, e, w1, w2, w3, aux)


def _node_body(h_ref, p0_ref, p1_ref, wh_ref, wa_ref, w2_ref, w3_ref, aux_ref,
               wd_ref, ws_ref, o_ref, a_ref, b_ref):
    b1, b2, b3 = aux_ref[0:1, :], aux_ref[1:2, :], aux_ref[2:3, :]
    g, be = aux_ref[3:4, :], aux_ref[4:5, :]
    h = h_ref[...]
    agg = p0_ref[...] + p1_ref[...]
    t = _dot(h, wh_ref[...]) + _dot(agg, wa_ref[...]) + b1
    t = jnp.maximum(t, 0.0)
    t = jnp.maximum(_dot(t, w2_ref[...]) + b2, 0.0)
    t = _dot(t, w3_ref[...]) + b3
    hn = h + _ln(t, g, be)
    o_ref[...] = hn
    a_ref[...] = _dot(hn, wd_ref[...])
    b_ref[...] = _dot(hn, ws_ref[...])


def _node_mlp(h, p0, p1, wh, wa, w2, w3, aux, wd, ws):
    return pl.pallas_call(
        _node_body,
        grid=(N // BRN,),
        in_specs=[
            pl.BlockSpec((BRN, H), lambda i: (i, 0)),
            pl.BlockSpec((BRN, H), lambda i: (i, 0)),
            pl.BlockSpec((BRN, H), lambda i: (i, 0)),
            pl.BlockSpec((H, H), lambda i: (0, 0)),
            pl.BlockSpec((H, H), lambda i: (0, 0)),
            pl.BlockSpec((H, H), lambda i: (0, 0)),
            pl.BlockSpec((H, H), lambda i: (0, 0)),
            pl.BlockSpec((8, H), lambda i: (0, 0)),
            pl.BlockSpec((H, H), lambda i: (0, 0)),
            pl.BlockSpec((H, H), lambda i: (0, 0)),
        ],
        out_specs=[
            pl.BlockSpec((BRN, H), lambda i: (i, 0)),
            pl.BlockSpec((BRN, H), lambda i: (i, 0)),
            pl.BlockSpec((BRN, H), lambda i: (i, 0)),
        ],
        out_shape=[
            jax.ShapeDtypeStruct((N, H), jnp.float32),
            jax.ShapeDtypeStruct((N, H), jnp.float32),
            jax.ShapeDtypeStruct((N, H), jnp.float32),
        ],
    )(h, p0, p1, wh, wa, w2, w3, aux, wd, ws)


def _dec_body(h_ref, w1_ref, w2_ref, w3_ref, aux_ref, o_ref):
    b1, b2, b3 = aux_ref[0:1, :], aux_ref[1:2, :], aux_ref[2:3, :]
    t = jnp.maximum(_dot(h_ref[...], w1_ref[...]) + b1, 0.0)
    t = jnp.maximum(_dot(t, w2_ref[...]) + b2, 0.0)
    o_ref[...] = _dot(t, w3_ref[...]) + b3


def _dec(h, w1, w2, w3, aux):
    return pl.pallas_call(
        _dec_body,
        grid=(N // BRN,),
        in_specs=[
            pl.BlockSpec((BRN, H), lambda i: (i, 0)),
            pl.BlockSpec((H, H), lambda i: (0, 0)),
            pl.BlockSpec((H, H), lambda i: (0, 0)),
            pl.BlockSpec((H, H), lambda i: (0, 0)),
            pl.BlockSpec((8, H), lambda i: (0, 0)),
        ],
        out_specs=pl.BlockSpec((BRN, H), lambda i: (i, 0)),
        out_shape=jax.ShapeDtypeStruct((N, H), jnp.float32),
    )(h, w1, w2, w3, aux)


def _sc_gather_add(a, b, dst, src):
    mesh = plsc.VectorSubcoreMesh(core_axis_name="c", subcore_axis_name="s")
    LA = NSL - 1

    @functools.partial(
        pl.kernel,
        mesh=mesh,
        out_type=jax.ShapeDtypeStruct((E, H), jnp.float32),
        scratch_types=(
            pltpu.VMEM((EW,), jnp.int32),
            pltpu.VMEM((EW,), jnp.int32),
            pltpu.VMEM((NSL, CH, H), jnp.float32),
            pltpu.VMEM((NSL, CH, H), jnp.float32),
            pltpu.SemaphoreType.DMA((NSL,)),
            pltpu.SemaphoreType.DMA((NSL,)),
            pltpu.SemaphoreType.DMA((NSL,)),
        ),
    )
    def k(a_hbm, b_hbm, d_hbm, s_hbm, g_hbm, idxd, idxs, abuf, bbuf, sga, sgb, sw):
        w = lax.axis_index("s") * NC + lax.axis_index("c")
        ebase = w * EW
        pltpu.sync_copy(d_hbm.at[pl.ds(ebase, EW)], idxd)
        pltpu.sync_copy(s_hbm.at[pl.ds(ebase, EW)], idxs)

        def start(j, slot):
            pltpu.async_copy(a_hbm.at[idxd.at[pl.ds(j * CH, CH)]],
                             abuf.at[slot], sga.at[slot])
            pltpu.async_copy(b_hbm.at[idxs.at[pl.ds(j * CH, CH)]],
                             bbuf.at[slot], sgb.at[slot])

        def finish(j, slot):
            pltpu.make_async_copy(a_hbm.at[idxd.at[pl.ds(0, CH)]],
                                  abuf.at[slot], sga.at[slot]).wait()
            pltpu.make_async_copy(b_hbm.at[idxs.at[pl.ds(0, CH)]],
                                  bbuf.at[slot], sgb.at[slot]).wait()

            def addrows(r, carry):
                for rr in range(8):
                    for t in range(H // 16):
                        s_ = pl.ds(t * 16, 16)
                        plsc.addupdate(abuf.at[slot, r * 8 + rr, s_],
                                       bbuf[slot, r * 8 + rr, s_])
                return carry

            lax.fori_loop(0, CH // 8, addrows, 0)
            pltpu.async_copy(abuf.at[slot], g_hbm.at[pl.ds(ebase + j * CH, CH)],
                             sw.at[slot])

        def drain_w(slot):
            pltpu.make_async_copy(abuf.at[slot], g_hbm.at[pl.ds(0, CH)],
                                  sw.at[slot]).wait()

        for j0 in range(LA):
            start(j0, j0)

        def body(j, carry):
            for sl in range(NSL):
                @pl.when(j % NSL == sl)
                def _step(sl=sl):
                    @pl.when(j >= NSL)
                    def _dw():
                        drain_w(sl)
                    start(j, sl)
                    finish(j - LA, (sl + 1) % NSL)
            return carry

        lax.fori_loop(LA, NCH, body, 0)
        for jt in range(NCH - LA, NCH):
            finish(jt, jt % NSL)
        for sl in range(NSL):
            drain_w(sl)

    return k(a, b, dst, src)


def _sc_scatter(e, src2dp):
    mesh = plsc.VectorSubcoreMesh(core_axis_name="c", subcore_axis_name="s")

    @functools.partial(
        pl.kernel,
        mesh=mesh,
        out_type=jax.ShapeDtypeStruct((NC * NP, H), jnp.float32),
        scratch_types=(
            pltpu.VMEM((NCH + 11, CH), jnp.int32),
            pltpu.VMEM((CH, H), jnp.float32),
            pltpu.VMEM((CH, H), jnp.float32),
            pltpu.VMEM((ZCH, H), jnp.float32),
            pltpu.VMEM_SHARED((NP, H), jnp.float32),
            pltpu.SemaphoreType.DMA,
            pltpu.SemaphoreType.DMA,
        ),
    )
    def k(e_hbm, s_hbm, o_hbm, idxv, rows0, rows1, obuf, acc, sl0, sl1):
        c = lax.axis_index("c")
        s = lax.axis_index("s")
        w = c * NS + s

        def zb(r, carry):
            for t in range(H // 16):
                obuf[r, pl.ds(t * 16, 16)] = jnp.zeros((16,), jnp.float32)
            return carry

        lax.fori_loop(0, ZCH, zb, 0)
        for t in range(NPT // ZCH):
            pltpu.sync_copy(obuf, acc.at[pl.ds(s * NPT + t * ZCH, ZCH)])
        plsc.subcore_barrier()

        rbase = w * NCH
        rb8 = (rbase // 8) * 8
        off = rbase - rb8
        pltpu.sync_copy(s_hbm.at[pl.ds(rb8, NCH + 11)], idxv)

        rows, sl = (rows0, rows1), (sl0, sl1)

        def start(j, slot):
            pltpu.async_copy(e_hbm.at[pl.ds(w * EW + j * CH, CH)], rows[slot], sl[slot])

        def add(j, slot):
            pltpu.make_async_copy(e_hbm.at[pl.ds(0, CH)], rows[slot], sl[slot]).wait()
            pltpu.sync_copy(rows[slot], acc.at[idxv.at[off + j]], add=True)

        start(0, 0)

        def body(j, carry):
            @pl.when(j % 2 == 1)
            def odd():
                start(j, 1)
                add(j - 1, 0)

            @pl.when(j % 2 == 0)
            def even():
                start(j, 0)
                add(j - 1, 1)

            return carry

        lax.fori_loop(1, NCH, body, 0)
        add(NCH - 1, 0)
        plsc.subcore_barrier()

        for t in range(NPT // ZCH):
            pltpu.sync_copy(acc.at[pl.ds(s * NPT + t * ZCH, ZCH)], obuf)
            pltpu.sync_copy(obuf, o_hbm.at[pl.ds(c * NP + s * NPT + t * ZCH, ZCH)])

    return k(e, src2dp)


def _aux(b1, b2, b3, g=None, be=None):
    z = jnp.zeros_like(b1)
    g = z if g is None else g
    be = z if be is None else be
    return jnp.stack([b1, b2, b3, g, be, z, z, z])


def kernel(x, edge_index, edge_attr, mean_vec_x, std_vec_x, mean_vec_edge,
           std_vec_edge, params):
    p = params
    f32 = jnp.float32

    w1n = p['ne_W1'] / std_vec_x[:, None]
    b1n = p['ne_b1'] - (mean_vec_x / std_vec_x) @ p['ne_W1']
    w1n = jnp.pad(w1n, ((0, 16 - w1n.shape[0]), (0, 0)))
    xp = jnp.pad(x.astype(f32), ((0, 0), (0, 16 - x.shape[1])))
    h = _mlp3(xp, w1n, p['ne_W2'], p['ne_W3'],
              _aux(b1n, p['ne_b2'], p['ne_b3'], p['ne_g'], p['ne_be']), BRN)

    w1e = p['ee_W1'] / std_vec_edge[:, None]
    b1e = p['ee_b1'] - (mean_vec_edge / std_vec_edge) @ p['ee_W1']
    w1e = jnp.pad(w1e, ((0, 8 - w1e.shape[0]), (0, 0)))
    eap = jnp.pad(edge_attr.astype(f32), ((0, 0), (0, 8 - edge_attr.shape[1])))
    e = _mlp3(eap, w1e, p['ee_W2'], p['ee_W3'],
              _aux(b1e, p['ee_b2'], p['ee_b3'], p['ee_g'], p['ee_be']), BRE)

    src = edge_index[0].astype(jnp.int32)
    dst = edge_index[1].astype(jnp.int32)
    src2dp = jnp.pad(src.reshape(E // CH, CH), ((0, 8), (0, 0)))

    a = b = None
    for l in range(L):
        w1 = p['ep_W1'][l]
        if l == 0:
            a, b = _ab(h, w1[:H], w1[H:2 * H])
        g = _sc_gather_add(a, b, dst, src)
        e = _edge_mlp(g, e, w1[2 * H:], p['ep_W2'][l], p['ep_W3'][l],
                      _aux(p['ep_b1'][l], p['ep_b2'][l], p['ep_b3'][l],
                           p['ep_g'][l], p['ep_be'][l]))
        part = _sc_scatter(e, src2dp)
        nw1 = p['np_W1'][l]
        w1nx = p['ep_W1'][l + 1] if l + 1 < L else p['ep_W1'][l]
        h, a, b = _node_mlp(h, part[:N], part[NP:NP + N], nw1[:H], nw1[H:],
                            p['np_W2'][l], p['np_W3'][l],
                            _aux(p['np_b1'][l], p['np_b2'][l], p['np_b3'][l],
                                 p['np_g'][l], p['np_be'][l]),
                            w1nx[:H], w1nx[H:2 * H])

    w3p = jnp.pad(p['dec_W3'], ((0, 0), (0, H - p['dec_W3'].shape[1])))
    b3p = jnp.pad(p['dec_b3'], ((0, H - p['dec_b3'].shape[0]),))
    out = _dec(h, p['dec_W1'], p['dec_W2'], w3p,
               _aux(p['dec_b1'], p['dec_b2'], b3p))
    return out[:, :2]
